# baseline (speedup 1.0000x reference)
import functools

import jax
import jax.numpy as jnp
from jax import lax
from jax.experimental import pallas as pl
from jax.experimental.pallas import tpu as pltpu
from jax.experimental.pallas import tpu_sc as plsc

N = 20000
P = 20480
NT = 16
SHARD = P // NT
VPT = SHARD // 16
CAP = 128
ACAP = SHARD + 16
AGRP = ACAP // 16
S = 6
TBLK = 64
POOLG = NT * S // 16
SELMAX = 10
IOU_THRES = 0.45
CONF_THRES = 0.25
MAX_DET = 300
NEG = -1.0
BIG = 1 << 30


def _nms_body(x1_hbm, y1_hbm, x2_hbm, y2_hbm, sc_hbm, out_hbm,
              lx1, ly1, lx2, ly2, lsc,
              ax1, ay1, ax2, ay2, aact, agidx,
              pub, recv, poolact, pgid, px1, py1, px2, py2, keptb, recs_sh):
    t = lax.axis_index("s")
    base = t * SHARD
    lane = lax.iota(jnp.int32, 16)
    zeros16f = jnp.zeros((16,), jnp.float32)
    neg16f = jnp.full((16,), NEG, jnp.float32)
    lane0 = lane == 0

    pltpu.sync_copy(x1_hbm.at[pl.ds(base, SHARD)], lx1)
    pltpu.sync_copy(y1_hbm.at[pl.ds(base, SHARD)], ly1)
    pltpu.sync_copy(x2_hbm.at[pl.ds(base, SHARD)], lx2)
    pltpu.sync_copy(y2_hbm.at[pl.ds(base, SHARD)], ly2)
    pltpu.sync_copy(sc_hbm.at[pl.ds(base, SHARD)], lsc)

    def zbody(j, _):
        keptb[pl.ds(j * 16, 16)] = zeros16f
        return 0
    lax.fori_loop(0, MAX_DET, zbody, 0)

    def mc_body(j, c):
        vm, vc = c
        s = lsc[pl.ds(j * 16, 16)]
        return jnp.maximum(vm, s), vc + (s > CONF_THRES).astype(jnp.float32)

    vm, vc = lax.fori_loop(0, VPT, mc_body,
                           (jnp.full((16,), -2.0, jnp.float32), zeros16f))
    maxsc = jnp.max(vm)
    cnt_conf = jnp.sum(vc)

    def bs_body(it, c):
        lo, hi = c
        mid = (lo + hi) * 0.5

        def cb(j, a):
            s = lsc[pl.ds(j * 16, 16)]
            return a + (s > mid).astype(jnp.float32)

        cnt = jnp.sum(lax.fori_loop(0, VPT, cb, zeros16f))
        big = cnt > float(CAP)
        return jnp.where(big, mid, lo), jnp.where(big, hi, mid)

    _, hi = lax.fori_loop(0, 16, bs_body,
                          (jnp.float32(CONF_THRES), maxsc + 1.0))
    T = jnp.where(cnt_conf <= float(CAP), jnp.float32(CONF_THRES), hi)

    def prefill(j, _):
        o = j * 16
        aact[pl.ds(o, 16)] = neg16f
        agidx[pl.ds(o, 16)] = jnp.full((16,), -7, jnp.int32)
        return 0

    def compact(thresh):
        lax.fori_loop(0, AGRP, prefill, 0)

        def cp(j, off):
            o = j * 16
            s = lsc[pl.ds(o, 16)]
            mask = s > thresh
            mi = mask.astype(jnp.int32)
            cs = plsc.cumsum(mi)
            pos = off + cs - mi
            plsc.store_scatter(aact, [pos], s, mask=mask)
            plsc.store_scatter(ax1, [pos], lx1[pl.ds(o, 16)], mask=mask)
            plsc.store_scatter(ay1, [pos], ly1[pl.ds(o, 16)], mask=mask)
            plsc.store_scatter(ax2, [pos], lx2[pl.ds(o, 16)], mask=mask)
            plsc.store_scatter(ay2, [pos], ly2[pl.ds(o, 16)], mask=mask)
            plsc.store_scatter(agidx, [pos], base + o + lane, mask=mask)
            return off + jnp.max(cs)

        return lax.fori_loop(0, VPT, cp, jnp.int32(0))

    cnt0 = compact(T)
    ng0 = (cnt0 + 15) // 16
    more = cnt_conf > cnt0.astype(jnp.float32)

    def active_argmax(ngroups):
        def am(g, c):
            vmx, vix = c
            a = aact[pl.ds(g * 16, 16)]
            m = a > vmx
            return jnp.where(m, a, vmx), jnp.where(m, g * 16 + lane, vix)

        vmx, vix = lax.fori_loop(0, ngroups, am,
                                 (jnp.full((16,), -2.0, jnp.float32),
                                  jnp.zeros((16,), jnp.int32)))
        gm = jnp.max(vmx)
        sp = jnp.min(jnp.where(vmx == gm, vix, BIG))
        return gm, sp

    def active_argmax_static(_):
        vmx = jnp.full((16,), -2.0, jnp.float32)
        vix = jnp.zeros((16,), jnp.int32)
        for g in range(CAP // 16):
            a = aact[pl.ds(g * 16, 16)]
            m = a > vmx
            vmx = jnp.where(m, a, vmx)
            vix = jnp.where(m, g * 16 + lane, vix)
        gm = jnp.max(vmx)
        sp = jnp.min(jnp.where(vmx == gm, vix, BIG))
        return gm, sp

    def round_body(k, carry):
        def do_round():
            _, ng0_, refilled0, nsel0 = carry

            gm0, sp0 = lax.cond(refilled0 == 0, active_argmax_static,
                                active_argmax, ng0_)

            def refill():
                cnt = compact(jnp.float32(CONF_THRES))
                ngr = (cnt + 15) // 16

                def kf(j, _):
                    def fldk(c):
                        return plsc.load_gather(
                            keptb, [jnp.full((16,), j * 16 + c, jnp.int32)])

                    kx1, ky1, kx2, ky2 = fldk(0), fldk(1), fldk(2), fldk(3)
                    ka = (kx2 - kx1) * (ky2 - ky1)

                    def kg(g, _2):
                        o = g * 16
                        x1v = ax1[pl.ds(o, 16)]
                        y1v = ay1[pl.ds(o, 16)]
                        x2v = ax2[pl.ds(o, 16)]
                        y2v = ay2[pl.ds(o, 16)]
                        actv = aact[pl.ds(o, 16)]
                        av = (x2v - x1v) * (y2v - y1v)
                        xx1 = jnp.maximum(x1v, kx1)
                        yy1 = jnp.maximum(y1v, ky1)
                        xx2 = jnp.minimum(x2v, kx2)
                        yy2 = jnp.minimum(y2v, ky2)
                        inter = (jnp.maximum(xx2 - xx1, 0.0)
                                 * jnp.maximum(yy2 - yy1, 0.0))
                        denom = av + ka - inter + 1e-9
                        kill = inter > IOU_THRES * denom
                        aact[pl.ds(o, 16)] = jnp.where(kill, NEG, actv)
                        return 0

                    lax.fori_loop(0, ngr, kg, 0)
                    return 0

                lax.fori_loop(0, nsel0, kf, 0)
                gm2, sp2 = active_argmax(ngr)
                return gm2, sp2, ngr, jnp.int32(1)

            need = (gm0 < -0.5) & more & (refilled0 == 0)
            gm0, sp0, ng, refilled = lax.cond(
                need, refill, lambda: (gm0, sp0, ng0_, refilled0))

            def extract(amax):
                out = []
                gm, sp = gm0, sp0
                for r in range(S):
                    out += [gm, sp]
                    plsc.store_scatter(aact,
                                       [jnp.full((16,), sp, jnp.int32)],
                                       neg16f, mask=lane0)
                    gm, sp = amax(ng)
                return tuple(out) + (gm,)

            ext = lax.cond(refilled == 0,
                           lambda: extract(active_argmax_static),
                           lambda: extract(active_argmax))
            recs = [(ext[2 * r], ext[2 * r + 1]) for r in range(S)]
            fifth = ext[2 * S]
            for (g_, s_) in recs:
                plsc.store_scatter(aact, [jnp.full((16,), s_, jnp.int32)],
                                   jnp.full((16,), g_, jnp.float32),
                                   mask=lane0)
            bt = jnp.maximum(fifth,
                             jnp.where(more & (refilled == 0), T,
                                       jnp.float32(NEG)))

            flds = []
            for (g_, s_) in recs:
                spv = jnp.full((16,), s_, jnp.int32)
                flds.append((jnp.full((16,), g_, jnp.float32),
                             plsc.load_gather(agidx, [spv])
                                 .astype(jnp.float32),
                             plsc.load_gather(ax1, [spv]),
                             plsc.load_gather(ay1, [spv]),
                             plsc.load_gather(ax2, [spv]),
                             plsc.load_gather(ay2, [spv])))

            def pair(a, b):
                v = zeros16f
                for c in range(5, -1, -1):
                    v = jnp.where((lane & 7) == c,
                                  jnp.where(lane < 8, a[c], b[c]), v)
                return v

            for i in range(S // 2):
                pub[pl.ds(i * 16, 16)] = pair(flds[2 * i], flds[2 * i + 1])
            pub[pl.ds(S * 8, 16)] = jnp.where(
                lane0, jnp.full((16,), bt, jnp.float32), zeros16f)
            par = (k % 2) * (NT * TBLK)
            pltpu.sync_copy(pub, recs_sh.at[pl.ds(par + t * TBLK, TBLK)])
            plsc.subcore_barrier()
            pltpu.sync_copy(recs_sh.at[pl.ds(par, NT * TBLK)], recv)

            bvals = plsc.load_gather(recv, [lane * TBLK + S * 8])
            B = jnp.max(bvals)
            addrs = []
            for g in range(POOLG):
                e = g * 16 + lane
                addrs.append((e // S) * TBLK + (e % S) * 8)
            for g in range(POOLG):
                o = g * 16
                poolact[pl.ds(o, 16)] = plsc.load_gather(recv, [addrs[g]])
                pgid[pl.ds(o, 16)] = plsc.load_gather(recv, [addrs[g] + 1])
                px1[pl.ds(o, 16)] = plsc.load_gather(recv, [addrs[g] + 2])
                py1[pl.ds(o, 16)] = plsc.load_gather(recv, [addrs[g] + 3])
                px2[pl.ds(o, 16)] = plsc.load_gather(recv, [addrs[g] + 4])
                py2[pl.ds(o, 16)] = plsc.load_gather(recv, [addrs[g] + 5])

            def sel_body(si, sc):
                go, ns = sc

                def try_sel():
                    vmx = jnp.full((16,), -2.0, jnp.float32)
                    vix = jnp.zeros((16,), jnp.int32)
                    for g in range(POOLG):
                        a = poolact[pl.ds(g * 16, 16)]
                        m = a > vmx
                        vmx = jnp.where(m, a, vmx)
                        vix = jnp.where(m, g * 16 + lane, vix)
                    pm = jnp.max(vmx)
                    pp = jnp.min(jnp.where(vmx == pm, vix, BIG))
                    valid = ((pm > CONF_THRES) & (ns < MAX_DET)
                             & ((si == 0) | (pm > B)))

                    def do_sel():
                        pa = (pp // S) * TBLK + (pp % S) * 8

                        def rf(c):
                            return plsc.load_gather(
                                recv, [jnp.full((16,), pa + c, jnp.int32)])

                        wsc, wgf = rf(0), rf(1)
                        wx1, wy1, wx2, wy2 = rf(2), rf(3), rf(4), rf(5)
                        row = jnp.where(lane == 0, wx1,
                              jnp.where(lane == 1, wy1,
                              jnp.where(lane == 2, wx2,
                              jnp.where(lane == 3, wy2,
                              jnp.where(lane == 4, wsc, zeros16f)))))
                        keptb[pl.ds(ns * 16, 16)] = row
                        wa = (wx2 - wx1) * (wy2 - wy1)

                        for g in range(POOLG):
                            o = g * 16
                            x1p = px1[pl.ds(o, 16)]
                            y1p = py1[pl.ds(o, 16)]
                            x2p = px2[pl.ds(o, 16)]
                            y2p = py2[pl.ds(o, 16)]
                            gp = pgid[pl.ds(o, 16)]
                            ap = (x2p - x1p) * (y2p - y1p)
                            xx1 = jnp.maximum(x1p, wx1)
                            yy1 = jnp.maximum(y1p, wy1)
                            xx2 = jnp.minimum(x2p, wx2)
                            yy2 = jnp.minimum(y2p, wy2)
                            inter = (jnp.maximum(xx2 - xx1, 0.0)
                                     * jnp.maximum(yy2 - yy1, 0.0))
                            denom = ap + wa - inter + 1e-9
                            killp = (inter > IOU_THRES * denom) | (gp == wgf)
                            pav = poolact[pl.ds(g * 16, 16)]
                            poolact[pl.ds(g * 16, 16)] = jnp.where(
                                killp, NEG, pav)

                        wgi = wgf.astype(jnp.int32)

                        def ab(g, _):
                            o = g * 16
                            x1v = ax1[pl.ds(o, 16)]
                            y1v = ay1[pl.ds(o, 16)]
                            x2v = ax2[pl.ds(o, 16)]
                            y2v = ay2[pl.ds(o, 16)]
                            actv = aact[pl.ds(o, 16)]
                            gidxv = agidx[pl.ds(o, 16)]
                            av = (x2v - x1v) * (y2v - y1v)
                            xx1 = jnp.maximum(x1v, wx1)
                            yy1 = jnp.maximum(y1v, wy1)
                            xx2 = jnp.minimum(x2v, wx2)
                            yy2 = jnp.minimum(y2v, wy2)
                            inter = (jnp.maximum(xx2 - xx1, 0.0)
                                     * jnp.maximum(yy2 - yy1, 0.0))
                            denom = av + wa - inter + 1e-9
                            kill = ((inter > IOU_THRES * denom)
                                    | (gidxv == wgi))
                            aact[pl.ds(o, 16)] = jnp.where(kill, NEG, actv)
                            return 0

                        def ab_static():
                            for g in range(CAP // 16):
                                ab(g, 0)
                            return 0

                        lax.cond(refilled == 0, ab_static,
                                 lambda: lax.fori_loop(0, ng, ab, 0))
                        return jnp.int32(1), ns + 1

                    return lax.cond(valid, do_sel,
                                    lambda: (jnp.int32(0), ns))

                return lax.cond(go == 1, try_sel,
                                lambda: (jnp.int32(0), ns))

            _, nsel2 = lax.fori_loop(0, SELMAX, sel_body,
                                     (jnp.int32(1), nsel0))
            cont = ((nsel2 > nsel0) & (nsel2 < MAX_DET)).astype(jnp.int32)
            return cont, ng, refilled, nsel2

        def skip():
            return (jnp.int32(0),) + tuple(carry[1:])

        return lax.cond(carry[0] == 1, do_round, skip)

    init = (jnp.int32(1), ng0, jnp.int32(0), jnp.int32(0))
    lax.fori_loop(0, MAX_DET, round_body, init)

    @pl.when(t == 0)
    def _flush():
        pltpu.sync_copy(keptb, out_hbm)


@functools.partial(
    pl.kernel,
    out_type=jax.ShapeDtypeStruct((MAX_DET * 16,), jnp.float32),
    mesh=plsc.VectorSubcoreMesh(core_axis_name="c", subcore_axis_name="s",
                                num_cores=1, num_subcores=16),
    compiler_params=pltpu.CompilerParams(needs_layout_passes=False),
    scratch_types=[
        pltpu.VMEM((SHARD,), jnp.float32),
        pltpu.VMEM((SHARD,), jnp.float32),
        pltpu.VMEM((SHARD,), jnp.float32),
        pltpu.VMEM((SHARD,), jnp.float32),
        pltpu.VMEM((SHARD,), jnp.float32),
        pltpu.VMEM((ACAP,), jnp.float32),
        pltpu.VMEM((ACAP,), jnp.float32),
        pltpu.VMEM((ACAP,), jnp.float32),
        pltpu.VMEM((ACAP,), jnp.float32),
        pltpu.VMEM((ACAP,), jnp.float32),
        pltpu.VMEM((ACAP,), jnp.int32),
        pltpu.VMEM((TBLK,), jnp.float32),
        pltpu.VMEM((NT * TBLK,), jnp.float32),
        pltpu.VMEM((NT * S,), jnp.float32),
        pltpu.VMEM((NT * S,), jnp.float32),
        pltpu.VMEM((NT * S,), jnp.float32),
        pltpu.VMEM((NT * S,), jnp.float32),
        pltpu.VMEM((NT * S,), jnp.float32),
        pltpu.VMEM((NT * S,), jnp.float32),
        pltpu.VMEM((MAX_DET * 16,), jnp.float32),
        pltpu.VMEM_SHARED((2 * NT * TBLK,), jnp.float32),
    ],
)
def _nms_sc(x1_hbm, y1_hbm, x2_hbm, y2_hbm, sc_hbm, out_hbm,
            lx1, ly1, lx2, ly2, lsc,
            ax1, ay1, ax2, ay2, aact, agidx,
            pub, recv, poolact, pgid, px1, py1, px2, py2, keptb, recs_sh):
    _nms_body(x1_hbm, y1_hbm, x2_hbm, y2_hbm, sc_hbm, out_hbm,
              lx1, ly1, lx2, ly2, lsc,
              ax1, ay1, ax2, ay2, aact, agidx,
              pub, recv, poolact, pgid, px1, py1, px2, py2, keptb, recs_sh)


def kernel(boxes, scores):
    pad = P - N
    x1 = jnp.pad(boxes[:, 0], (0, pad))
    y1 = jnp.pad(boxes[:, 1], (0, pad))
    x2 = jnp.pad(boxes[:, 2], (0, pad))
    y2 = jnp.pad(boxes[:, 3], (0, pad))
    sc = jnp.pad(scores, (0, pad), constant_values=NEG)
    flat = _nms_sc(x1, y1, x2, y2, sc)
    return flat.reshape(MAX_DET, 16)[:, :5]


# speedup vs baseline: 1.0827x; 1.0827x over previous
import functools

import jax
import jax.numpy as jnp
from jax import lax
from jax.experimental import pallas as pl
from jax.experimental.pallas import tpu as pltpu
from jax.experimental.pallas import tpu_sc as plsc

N = 20000
P = 20480
NT = 16
SHARD = P // NT
VPT = SHARD // 16
CAP = 128
ACAP = SHARD + 16
AGRP = ACAP // 16
S = 4
TBLK = 48
POOLG = NT * S // 16
SELMAX = 8
IOU_THRES = 0.45
CONF_THRES = 0.25
MAX_DET = 300
NEG = -1.0
BIG = 1 << 30


def _nms_body(x1_hbm, y1_hbm, x2_hbm, y2_hbm, sc_hbm, out_hbm,
              lx1, ly1, lx2, ly2, lsc,
              ax1, ay1, ax2, ay2, aact, agidx,
              pub, recv, poolact, pgid, px1, py1, px2, py2, keptb, recs_sh):
    t = lax.axis_index("s")
    base = t * SHARD
    lane = lax.iota(jnp.int32, 16)
    zeros16f = jnp.zeros((16,), jnp.float32)
    neg16f = jnp.full((16,), NEG, jnp.float32)
    lane0 = lane == 0

    pltpu.sync_copy(x1_hbm.at[pl.ds(base, SHARD)], lx1)
    pltpu.sync_copy(y1_hbm.at[pl.ds(base, SHARD)], ly1)
    pltpu.sync_copy(x2_hbm.at[pl.ds(base, SHARD)], lx2)
    pltpu.sync_copy(y2_hbm.at[pl.ds(base, SHARD)], ly2)
    pltpu.sync_copy(sc_hbm.at[pl.ds(base, SHARD)], lsc)

    def zbody(j, _):
        keptb[pl.ds(j * 16, 16)] = zeros16f
        return 0
    lax.fori_loop(0, MAX_DET, zbody, 0)

    def mc_body(j, c):
        vm, vc = c
        s = lsc[pl.ds(j * 16, 16)]
        return jnp.maximum(vm, s), vc + (s > CONF_THRES).astype(jnp.float32)

    vm, vc = lax.fori_loop(0, VPT, mc_body,
                           (jnp.full((16,), -2.0, jnp.float32), zeros16f))
    maxsc = jnp.max(vm)
    cnt_conf = jnp.sum(vc)

    def bs_body(it, c):
        lo, hi = c
        mid = (lo + hi) * 0.5

        def cb(j, a):
            s = lsc[pl.ds(j * 16, 16)]
            return a + (s > mid).astype(jnp.float32)

        cnt = jnp.sum(lax.fori_loop(0, VPT, cb, zeros16f))
        big = cnt > float(CAP)
        return jnp.where(big, mid, lo), jnp.where(big, hi, mid)

    _, hi = lax.fori_loop(0, 16, bs_body,
                          (jnp.float32(CONF_THRES), maxsc + 1.0))
    T = jnp.where(cnt_conf <= float(CAP), jnp.float32(CONF_THRES), hi)

    def prefill(j, _):
        o = j * 16
        aact[pl.ds(o, 16)] = neg16f
        agidx[pl.ds(o, 16)] = jnp.full((16,), -7, jnp.int32)
        return 0

    def compact(thresh):
        lax.fori_loop(0, AGRP, prefill, 0)

        def cp(j, off):
            o = j * 16
            s = lsc[pl.ds(o, 16)]
            mask = s > thresh
            mi = mask.astype(jnp.int32)
            cs = plsc.cumsum(mi)
            pos = off + cs - mi
            plsc.store_scatter(aact, [pos], s, mask=mask)
            plsc.store_scatter(ax1, [pos], lx1[pl.ds(o, 16)], mask=mask)
            plsc.store_scatter(ay1, [pos], ly1[pl.ds(o, 16)], mask=mask)
            plsc.store_scatter(ax2, [pos], lx2[pl.ds(o, 16)], mask=mask)
            plsc.store_scatter(ay2, [pos], ly2[pl.ds(o, 16)], mask=mask)
            plsc.store_scatter(agidx, [pos], base + o + lane, mask=mask)
            return off + jnp.max(cs)

        return lax.fori_loop(0, VPT, cp, jnp.int32(0))

    cnt0 = compact(T)
    ng0 = (cnt0 + 15) // 16
    more = cnt_conf > cnt0.astype(jnp.float32)

    def active_argmax(ngroups):
        def am(g, c):
            vmx, vix = c
            a = aact[pl.ds(g * 16, 16)]
            m = a > vmx
            return jnp.where(m, a, vmx), jnp.where(m, g * 16 + lane, vix)

        vmx, vix = lax.fori_loop(0, ngroups, am,
                                 (jnp.full((16,), -2.0, jnp.float32),
                                  jnp.zeros((16,), jnp.int32)))
        gm = jnp.max(vmx)
        sp = jnp.min(jnp.where(vmx == gm, vix, BIG))
        return gm, sp

    def active_argmax_static(_):
        vmx = jnp.full((16,), -2.0, jnp.float32)
        vix = jnp.zeros((16,), jnp.int32)
        for g in range(CAP // 16):
            a = aact[pl.ds(g * 16, 16)]
            m = a > vmx
            vmx = jnp.where(m, a, vmx)
            vix = jnp.where(m, g * 16 + lane, vix)
        gm = jnp.max(vmx)
        sp = jnp.min(jnp.where(vmx == gm, vix, BIG))
        return gm, sp

    def round_body(k, carry):
        def do_round():
            _, ng0_, refilled0, nsel0 = carry

            gm0, sp0 = lax.cond(refilled0 == 0, active_argmax_static,
                                active_argmax, ng0_)

            def refill():
                cnt = compact(jnp.float32(CONF_THRES))
                ngr = (cnt + 15) // 16

                def kf(j, _):
                    def fldk(c):
                        return plsc.load_gather(
                            keptb, [jnp.full((16,), j * 16 + c, jnp.int32)])

                    kx1, ky1, kx2, ky2 = fldk(0), fldk(1), fldk(2), fldk(3)
                    ka = (kx2 - kx1) * (ky2 - ky1)

                    def kg(g, _2):
                        o = g * 16
                        x1v = ax1[pl.ds(o, 16)]
                        y1v = ay1[pl.ds(o, 16)]
                        x2v = ax2[pl.ds(o, 16)]
                        y2v = ay2[pl.ds(o, 16)]
                        actv = aact[pl.ds(o, 16)]
                        av = (x2v - x1v) * (y2v - y1v)
                        xx1 = jnp.maximum(x1v, kx1)
                        yy1 = jnp.maximum(y1v, ky1)
                        xx2 = jnp.minimum(x2v, kx2)
                        yy2 = jnp.minimum(y2v, ky2)
                        inter = (jnp.maximum(xx2 - xx1, 0.0)
                                 * jnp.maximum(yy2 - yy1, 0.0))
                        denom = av + ka - inter + 1e-9
                        kill = inter > IOU_THRES * denom
                        aact[pl.ds(o, 16)] = jnp.where(kill, NEG, actv)
                        return 0

                    lax.fori_loop(0, ngr, kg, 0)
                    return 0

                lax.fori_loop(0, nsel0, kf, 0)
                gm2, sp2 = active_argmax(ngr)
                return gm2, sp2, ngr, jnp.int32(1)

            need = (gm0 < -0.5) & more & (refilled0 == 0)
            gm0, sp0, ng, refilled = lax.cond(
                need, refill, lambda: (gm0, sp0, ng0_, refilled0))

            def extract(amax):
                out = []
                gm, sp = gm0, sp0
                for r in range(S):
                    out += [gm, sp]
                    plsc.store_scatter(aact,
                                       [jnp.full((16,), sp, jnp.int32)],
                                       neg16f, mask=lane0)
                    gm, sp = amax(ng)
                return tuple(out) + (gm,)

            ext = lax.cond(refilled == 0,
                           lambda: extract(active_argmax_static),
                           lambda: extract(active_argmax))
            recs = [(ext[2 * r], ext[2 * r + 1]) for r in range(S)]
            fifth = ext[2 * S]
            for (g_, s_) in recs:
                plsc.store_scatter(aact, [jnp.full((16,), s_, jnp.int32)],
                                   jnp.full((16,), g_, jnp.float32),
                                   mask=lane0)
            bt = jnp.maximum(fifth,
                             jnp.where(more & (refilled == 0), T,
                                       jnp.float32(NEG)))

            flds = []
            for (g_, s_) in recs:
                spv = jnp.full((16,), s_, jnp.int32)
                flds.append((jnp.full((16,), g_, jnp.float32),
                             plsc.load_gather(agidx, [spv])
                                 .astype(jnp.float32),
                             plsc.load_gather(ax1, [spv]),
                             plsc.load_gather(ay1, [spv]),
                             plsc.load_gather(ax2, [spv]),
                             plsc.load_gather(ay2, [spv])))

            def pair(a, b):
                v = zeros16f
                for c in range(5, -1, -1):
                    v = jnp.where((lane & 7) == c,
                                  jnp.where(lane < 8, a[c], b[c]), v)
                return v

            pub[pl.ds(0, 16)] = pair(flds[0], flds[1])
            pub[pl.ds(16, 16)] = pair(flds[2], flds[3])
            pub[pl.ds(32, 16)] = jnp.where(lane0,
                                           jnp.full((16,), bt, jnp.float32),
                                           zeros16f)
            par = (k % 2) * (NT * TBLK)
            pltpu.sync_copy(pub, recs_sh.at[pl.ds(par + t * TBLK, TBLK)])
            plsc.subcore_barrier()
            pltpu.sync_copy(recs_sh.at[pl.ds(par, NT * TBLK)], recv)

            bvals = plsc.load_gather(recv, [lane * TBLK + 32])
            B = jnp.max(bvals)
            addrs = []
            for g in range(POOLG):
                e = g * 16 + lane
                addrs.append((e >> 2) * TBLK + (e & 3) * 8)
            for g in range(POOLG):
                o = g * 16
                poolact[pl.ds(o, 16)] = plsc.load_gather(recv, [addrs[g]])
                pgid[pl.ds(o, 16)] = plsc.load_gather(recv, [addrs[g] + 1])
                px1[pl.ds(o, 16)] = plsc.load_gather(recv, [addrs[g] + 2])
                py1[pl.ds(o, 16)] = plsc.load_gather(recv, [addrs[g] + 3])
                px2[pl.ds(o, 16)] = plsc.load_gather(recv, [addrs[g] + 4])
                py2[pl.ds(o, 16)] = plsc.load_gather(recv, [addrs[g] + 5])

            def sel_body(si, sc):
                go, ns = sc

                def try_sel():
                    vmx = jnp.full((16,), -2.0, jnp.float32)
                    vix = jnp.zeros((16,), jnp.int32)
                    for g in range(POOLG):
                        a = poolact[pl.ds(g * 16, 16)]
                        m = a > vmx
                        vmx = jnp.where(m, a, vmx)
                        vix = jnp.where(m, g * 16 + lane, vix)
                    pm = jnp.max(vmx)
                    pp = jnp.min(jnp.where(vmx == pm, vix, BIG))
                    valid = ((pm > CONF_THRES) & (ns < MAX_DET)
                             & ((si == 0) | (pm > B)))

                    def do_sel():
                        pa = (pp >> 2) * TBLK + (pp & 3) * 8

                        def rf(c):
                            return plsc.load_gather(
                                recv, [jnp.full((16,), pa + c, jnp.int32)])

                        wsc, wgf = rf(0), rf(1)
                        wx1, wy1, wx2, wy2 = rf(2), rf(3), rf(4), rf(5)
                        row = jnp.where(lane == 0, wx1,
                              jnp.where(lane == 1, wy1,
                              jnp.where(lane == 2, wx2,
                              jnp.where(lane == 3, wy2,
                              jnp.where(lane == 4, wsc, zeros16f)))))
                        keptb[pl.ds(ns * 16, 16)] = row
                        wa = (wx2 - wx1) * (wy2 - wy1)

                        for g in range(POOLG):
                            o = g * 16
                            x1p = px1[pl.ds(o, 16)]
                            y1p = py1[pl.ds(o, 16)]
                            x2p = px2[pl.ds(o, 16)]
                            y2p = py2[pl.ds(o, 16)]
                            gp = pgid[pl.ds(o, 16)]
                            ap = (x2p - x1p) * (y2p - y1p)
                            xx1 = jnp.maximum(x1p, wx1)
                            yy1 = jnp.maximum(y1p, wy1)
                            xx2 = jnp.minimum(x2p, wx2)
                            yy2 = jnp.minimum(y2p, wy2)
                            inter = (jnp.maximum(xx2 - xx1, 0.0)
                                     * jnp.maximum(yy2 - yy1, 0.0))
                            denom = ap + wa - inter + 1e-9
                            killp = (inter > IOU_THRES * denom) | (gp == wgf)
                            pav = poolact[pl.ds(g * 16, 16)]
                            poolact[pl.ds(g * 16, 16)] = jnp.where(
                                killp, NEG, pav)

                        wgi = wgf.astype(jnp.int32)

                        def ab(g, _):
                            o = g * 16
                            x1v = ax1[pl.ds(o, 16)]
                            y1v = ay1[pl.ds(o, 16)]
                            x2v = ax2[pl.ds(o, 16)]
                            y2v = ay2[pl.ds(o, 16)]
                            actv = aact[pl.ds(o, 16)]
                            gidxv = agidx[pl.ds(o, 16)]
                            av = (x2v - x1v) * (y2v - y1v)
                            xx1 = jnp.maximum(x1v, wx1)
                            yy1 = jnp.maximum(y1v, wy1)
                            xx2 = jnp.minimum(x2v, wx2)
                            yy2 = jnp.minimum(y2v, wy2)
                            inter = (jnp.maximum(xx2 - xx1, 0.0)
                                     * jnp.maximum(yy2 - yy1, 0.0))
                            denom = av + wa - inter + 1e-9
                            kill = ((inter > IOU_THRES * denom)
                                    | (gidxv == wgi))
                            aact[pl.ds(o, 16)] = jnp.where(kill, NEG, actv)
                            return 0

                        def ab_static():
                            for g in range(CAP // 16):
                                ab(g, 0)
                            return 0

                        lax.cond(refilled == 0, ab_static,
                                 lambda: lax.fori_loop(0, ng, ab, 0))
                        return jnp.int32(1), ns + 1

                    return lax.cond(valid, do_sel,
                                    lambda: (jnp.int32(0), ns))

                return lax.cond(go == 1, try_sel,
                                lambda: (jnp.int32(0), ns))

            _, nsel2 = lax.fori_loop(0, SELMAX, sel_body,
                                     (jnp.int32(1), nsel0))
            cont = ((nsel2 > nsel0) & (nsel2 < MAX_DET)).astype(jnp.int32)
            return cont, ng, refilled, nsel2

        def skip():
            return (jnp.int32(0),) + tuple(carry[1:])

        return lax.cond(carry[0] == 1, do_round, skip)

    init = (jnp.int32(1), ng0, jnp.int32(0), jnp.int32(0))
    lax.fori_loop(0, MAX_DET, round_body, init)

    @pl.when(t == 0)
    def _flush():
        pltpu.sync_copy(keptb, out_hbm)


@functools.partial(
    pl.kernel,
    out_type=jax.ShapeDtypeStruct((MAX_DET * 16,), jnp.float32),
    mesh=plsc.VectorSubcoreMesh(core_axis_name="c", subcore_axis_name="s",
                                num_cores=1, num_subcores=16),
    compiler_params=pltpu.CompilerParams(needs_layout_passes=False),
    scratch_types=[
        pltpu.VMEM((SHARD,), jnp.float32),
        pltpu.VMEM((SHARD,), jnp.float32),
        pltpu.VMEM((SHARD,), jnp.float32),
        pltpu.VMEM((SHARD,), jnp.float32),
        pltpu.VMEM((SHARD,), jnp.float32),
        pltpu.VMEM((ACAP,), jnp.float32),
        pltpu.VMEM((ACAP,), jnp.float32),
        pltpu.VMEM((ACAP,), jnp.float32),
        pltpu.VMEM((ACAP,), jnp.float32),
        pltpu.VMEM((ACAP,), jnp.float32),
        pltpu.VMEM((ACAP,), jnp.int32),
        pltpu.VMEM((TBLK,), jnp.float32),
        pltpu.VMEM((NT * TBLK,), jnp.float32),
        pltpu.VMEM((NT * S,), jnp.float32),
        pltpu.VMEM((NT * S,), jnp.float32),
        pltpu.VMEM((NT * S,), jnp.float32),
        pltpu.VMEM((NT * S,), jnp.float32),
        pltpu.VMEM((NT * S,), jnp.float32),
        pltpu.VMEM((NT * S,), jnp.float32),
        pltpu.VMEM((MAX_DET * 16,), jnp.float32),
        pltpu.VMEM_SHARED((2 * NT * TBLK,), jnp.float32),
    ],
)
def _nms_sc(x1_hbm, y1_hbm, x2_hbm, y2_hbm, sc_hbm, out_hbm,
            lx1, ly1, lx2, ly2, lsc,
            ax1, ay1, ax2, ay2, aact, agidx,
            pub, recv, poolact, pgid, px1, py1, px2, py2, keptb, recs_sh):
    _nms_body(x1_hbm, y1_hbm, x2_hbm, y2_hbm, sc_hbm, out_hbm,
              lx1, ly1, lx2, ly2, lsc,
              ax1, ay1, ax2, ay2, aact, agidx,
              pub, recv, poolact, pgid, px1, py1, px2, py2, keptb, recs_sh)


def kernel(boxes, scores):
    pad = P - N
    x1 = jnp.pad(boxes[:, 0], (0, pad))
    y1 = jnp.pad(boxes[:, 1], (0, pad))
    x2 = jnp.pad(boxes[:, 2], (0, pad))
    y2 = jnp.pad(boxes[:, 3], (0, pad))
    sc = jnp.pad(scores, (0, pad), constant_values=NEG)
    flat = _nms_sc(x1, y1, x2, y2, sc)
    return flat.reshape(MAX_DET, 16)[:, :5]
